# baseline (speedup 1.0000x reference)
import jax
import jax.numpy as jnp
from jax.experimental import pallas as pl
from jax.experimental.pallas import tpu as pltpu


def _block_rows(n):
    for b in (400, 200, 100, 8):
        if n % b == 0:
            return b
    return n


def _p0_body(x_ref, w1_ref, s1_ref):
    s1 = jnp.dot(x_ref[...], w1_ref[...], preferred_element_type=jnp.float32)
    s1_ref[...] = s1.astype(jnp.bfloat16)


def _p1_body(adj_ref, s1_ref, b1_ref, w2_ref, adj16_ref, x11_ref, s2_ref):
    a16 = adj_ref[...].astype(jnp.bfloat16)
    adj16_ref[...] = a16
    agg = jnp.dot(a16, s1_ref[...], preferred_element_type=jnp.float32)
    x11 = jax.nn.sigmoid(agg + b1_ref[...])
    x11_ref[...] = x11
    s2 = jnp.dot(x11, w2_ref[...], preferred_element_type=jnp.float32)
    s2_ref[...] = s2.astype(jnp.bfloat16)


def _p2_body(adj16_ref, s2_ref, b2_ref, x11_ref, wla_ref, wlb_ref, bl_ref,
             w3a_ref, w3b_ref, l1_ref, s3_ref):
    agg = jnp.dot(adj16_ref[...], s2_ref[...],
                  preferred_element_type=jnp.float32)
    x12b = jax.nn.sigmoid(agg + b2_ref[...])
    x11 = x11_ref[...]
    l1_ref[...] = (jnp.dot(x11, wla_ref[...],
                           preferred_element_type=jnp.float32)
                   + jnp.dot(x12b, wlb_ref[...],
                             preferred_element_type=jnp.float32)
                   + bl_ref[...])
    s3 = (jnp.dot(x11, w3a_ref[...], preferred_element_type=jnp.float32)
          + jnp.dot(x12b, w3b_ref[...], preferred_element_type=jnp.float32))
    s3_ref[...] = s3.astype(jnp.bfloat16)


def _p3_body(adj16_ref, s3_ref, b3_ref, w4_ref, s4_ref):
    agg = jnp.dot(adj16_ref[...], s3_ref[...],
                  preferred_element_type=jnp.float32)
    x21 = jax.nn.sigmoid(agg + b3_ref[...])
    s4 = jnp.dot(x21, w4_ref[...], preferred_element_type=jnp.float32)
    s4_ref[...] = s4.astype(jnp.bfloat16)


def _p4_body(adj16_ref, s4_ref, b4_ref, x11_ref, l1_ref, out_ref):
    agg = jnp.dot(adj16_ref[...], s4_ref[...],
                  preferred_element_type=jnp.float32)
    x22 = jax.nn.sigmoid(agg + b4_ref[...])
    out_ref[...] = jax.nn.sigmoid(x11_ref[...] + x22 * l1_ref[...])


def _full(shape):
    return pl.BlockSpec(shape, lambda i: (0,) * len(shape))


def _rows(bi, f):
    return pl.BlockSpec((bi, f), lambda i: (i, 0))


def kernel(x, adj, W1, b1, W2, b2, W3, b3, W4, b4, Wl, bl):
    n, feat = x.shape
    f1 = W1.shape[1]
    f2 = W2.shape[1]
    pad = f1 - f2
    bi = _block_rows(n)
    grid = (n // bi,)
    params = pltpu.CompilerParams(dimension_semantics=("parallel",))

    b1r = b1.reshape(1, -1)
    b2r = jnp.pad(b2.reshape(1, -1), ((0, 0), (0, pad)))
    b3r = jnp.pad(b3.reshape(1, -1), ((0, 0), (0, pad)))
    b4r = b4.reshape(1, -1)
    blr = bl.reshape(1, -1)
    w2c = jnp.pad(W2, ((0, 0), (0, pad)))
    wla = Wl[:f1]
    wlb = jnp.pad(Wl[f1:], ((0, pad), (0, 0)))
    w3a = jnp.pad(W3[:f1], ((0, 0), (0, pad)))
    w3b = jnp.pad(W3[f1:], ((0, pad), (0, pad)))
    w4c = jnp.pad(W4, ((0, pad), (0, 0)))

    s1 = pl.pallas_call(
        _p0_body,
        out_shape=jax.ShapeDtypeStruct((n, f1), jnp.bfloat16),
    )(x, W1)

    adj16, x11, s2 = pl.pallas_call(
        _p1_body,
        grid=grid,
        in_specs=[_rows(bi, n), _full((n, f1)), _full((1, f1)),
                  _full((f1, f1))],
        out_specs=[_rows(bi, n), _rows(bi, f1), _rows(bi, f1)],
        out_shape=[jax.ShapeDtypeStruct((n, n), jnp.bfloat16),
                   jax.ShapeDtypeStruct((n, f1), jnp.float32),
                   jax.ShapeDtypeStruct((n, f1), jnp.bfloat16)],
        compiler_params=params,
    )(adj, s1, b1r, w2c)

    l1, s3 = pl.pallas_call(
        _p2_body,
        grid=grid,
        in_specs=[_rows(bi, n), _full((n, f1)), _full((1, f1)),
                  _rows(bi, f1), _full((f1, f1)), _full((f1, f1)),
                  _full((1, f1)), _full((f1, f1)), _full((f1, f1))],
        out_specs=[_rows(bi, f1), _rows(bi, f1)],
        out_shape=[jax.ShapeDtypeStruct((n, f1), jnp.float32),
                   jax.ShapeDtypeStruct((n, f1), jnp.bfloat16)],
        compiler_params=params,
    )(adj16, s2, b2r, x11, wla, wlb, blr, w3a, w3b)

    s4 = pl.pallas_call(
        _p3_body,
        grid=grid,
        in_specs=[_rows(bi, n), _full((n, f1)), _full((1, f1)),
                  _full((f1, f1))],
        out_specs=_rows(bi, f1),
        out_shape=jax.ShapeDtypeStruct((n, f1), jnp.bfloat16),
        compiler_params=params,
    )(adj16, s3, b3r, w4c)

    out = pl.pallas_call(
        _p4_body,
        grid=grid,
        in_specs=[_rows(bi, n), _full((n, f1)), _full((1, f1)),
                  _rows(bi, f1), _rows(bi, f1)],
        out_specs=_rows(bi, f1),
        out_shape=jax.ShapeDtypeStruct((n, f1), jnp.float32),
        compiler_params=params,
    )(adj16, s4, b4r, x11, l1)

    return out


# speedup vs baseline: 1.0740x; 1.0740x over previous
import jax
import jax.numpy as jnp
from jax.experimental import pallas as pl
from jax.experimental.pallas import tpu as pltpu


def _block_rows(n):
    for b in (400, 200, 100, 8):
        if n % b == 0:
            return b
    return n


def _p0_body(x_ref, w1_ref, s1_ref):
    s1 = jnp.dot(x_ref[...], w1_ref[...], preferred_element_type=jnp.float32)
    s1_ref[...] = s1.astype(jnp.bfloat16)


def _p1_body(adj_ref, s1_ref, b1_ref, w2_ref, adj16_ref, x11_ref, s2_ref):
    a16 = adj_ref[...].astype(jnp.bfloat16)
    adj16_ref[...] = a16
    agg = jnp.dot(a16, s1_ref[...], preferred_element_type=jnp.float32)
    x11 = jax.nn.sigmoid(agg + b1_ref[...])
    x11_ref[...] = x11
    s2 = jnp.dot(x11.astype(jnp.bfloat16), w2_ref[...],
                 preferred_element_type=jnp.float32)
    s2_ref[...] = s2.astype(jnp.bfloat16)


def _p2_body(adj16_ref, s2_ref, b2_ref, x11_ref, wcomb_ref, bl_ref,
             l1_ref, s3_ref):
    f1 = x11_ref.shape[1]
    agg = jnp.dot(adj16_ref[...], s2_ref[...],
                  preferred_element_type=jnp.float32)
    x12b = jax.nn.sigmoid(agg + b2_ref[...])
    xcat = jnp.concatenate(
        [x11_ref[...].astype(jnp.bfloat16), x12b.astype(jnp.bfloat16)],
        axis=1)
    r = jnp.dot(xcat, wcomb_ref[...], preferred_element_type=jnp.float32)
    l1_ref[...] = r[:, :f1] + bl_ref[...]
    s3_ref[...] = r[:, f1:].astype(jnp.bfloat16)


def _p3_body(adj16_ref, s3_ref, b3_ref, w4_ref, s4_ref):
    agg = jnp.dot(adj16_ref[...], s3_ref[...],
                  preferred_element_type=jnp.float32)
    x21 = jax.nn.sigmoid(agg + b3_ref[...]).astype(jnp.bfloat16)
    s4 = jnp.dot(x21, w4_ref[...], preferred_element_type=jnp.float32)
    s4_ref[...] = s4.astype(jnp.bfloat16)


def _p4_body(adj16_ref, s4_ref, b4_ref, x11_ref, l1_ref, out_ref):
    agg = jnp.dot(adj16_ref[...], s4_ref[...],
                  preferred_element_type=jnp.float32)
    x22 = jax.nn.sigmoid(agg + b4_ref[...])
    out_ref[...] = jax.nn.sigmoid(x11_ref[...] + x22 * l1_ref[...])


def _full(shape):
    return pl.BlockSpec(shape, lambda i: (0,) * len(shape))


def _rows(bi, f):
    return pl.BlockSpec((bi, f), lambda i: (i, 0))


def kernel(x, adj, W1, b1, W2, b2, W3, b3, W4, b4, Wl, bl):
    n, feat = x.shape
    f1 = W1.shape[1]
    f2 = W2.shape[1]
    pad = f1 - f2
    bi = _block_rows(n)
    grid = (n // bi,)
    bi2 = 1000 if n % 1000 == 0 else bi
    grid2 = (n // bi2,)
    params = pltpu.CompilerParams(dimension_semantics=("parallel",))

    b1r = b1.reshape(1, -1)
    b2r = jnp.pad(b2.reshape(1, -1), ((0, 0), (0, pad)))
    b3r = jnp.pad(b3.reshape(1, -1), ((0, 0), (0, pad)))
    b4r = b4.reshape(1, -1)
    blr = bl.reshape(1, -1)
    w2c = jnp.pad(W2, ((0, 0), (0, pad))).astype(jnp.bfloat16)
    wla = Wl[:f1]
    wlb = jnp.pad(Wl[f1:], ((0, pad), (0, 0)))
    w3a = jnp.pad(W3[:f1], ((0, 0), (0, pad)))
    w3b = jnp.pad(W3[f1:], ((0, pad), (0, pad)))
    wcomb = jnp.block([[wla, w3a], [wlb, w3b]]).astype(jnp.bfloat16)
    w4c = jnp.pad(W4, ((0, pad), (0, 0))).astype(jnp.bfloat16)

    s1 = pl.pallas_call(
        _p0_body,
        out_shape=jax.ShapeDtypeStruct((n, f1), jnp.bfloat16),
    )(x, W1)

    adj16, x11, s2 = pl.pallas_call(
        _p1_body,
        grid=grid,
        in_specs=[_rows(bi, n), _full((n, f1)), _full((1, f1)),
                  _full((f1, f1))],
        out_specs=[_rows(bi, n), _rows(bi, f1), _rows(bi, f1)],
        out_shape=[jax.ShapeDtypeStruct((n, n), jnp.bfloat16),
                   jax.ShapeDtypeStruct((n, f1), jnp.float32),
                   jax.ShapeDtypeStruct((n, f1), jnp.bfloat16)],
        compiler_params=params,
    )(adj, s1, b1r, w2c)

    l1, s3 = pl.pallas_call(
        _p2_body,
        grid=grid2,
        in_specs=[_rows(bi2, n), _full((n, f1)), _full((1, f1)),
                  _rows(bi2, f1), _full((2 * f1, 2 * f1)), _full((1, f1))],
        out_specs=[_rows(bi2, f1), _rows(bi2, f1)],
        out_shape=[jax.ShapeDtypeStruct((n, f1), jnp.float32),
                   jax.ShapeDtypeStruct((n, f1), jnp.bfloat16)],
        compiler_params=params,
    )(adj16, s2, b2r, x11, wcomb, blr)

    s4 = pl.pallas_call(
        _p3_body,
        grid=grid2,
        in_specs=[_rows(bi2, n), _full((n, f1)), _full((1, f1)),
                  _full((f1, f1))],
        out_specs=_rows(bi2, f1),
        out_shape=jax.ShapeDtypeStruct((n, f1), jnp.bfloat16),
        compiler_params=params,
    )(adj16, s3, b3r, w4c)

    out = pl.pallas_call(
        _p4_body,
        grid=grid2,
        in_specs=[_rows(bi2, n), _full((n, f1)), _full((1, f1)),
                  _rows(bi2, f1), _rows(bi2, f1)],
        out_specs=_rows(bi2, f1),
        out_shape=jax.ShapeDtypeStruct((n, f1), jnp.float32),
        compiler_params=params,
    )(adj16, s4, b4r, x11, l1)

    return out
